# baseline (speedup 1.0000x reference)
import jax
import jax.numpy as jnp
from jax.experimental import pallas as pl
from jax.experimental.pallas import tpu as pltpu

_NCHUNK = 8


def _policy_body(s_ref, x_ref, dsm_hbm, ap_ref, sv_ref, dsm_vmem, sems):
    n = dsm_vmem.shape[0]
    ch = n // _NCHUNK
    x = x_ref[...]

    def _copy(k):
        return pltpu.make_async_copy(
            dsm_hbm.at[pl.ds(k * ch, ch), :],
            dsm_vmem.at[pl.ds(k * ch, ch), :],
            sems.at[k],
        )

    for k in range(_NCHUNK):
        _copy(k).start()

    w_a1 = s_ref[0]
    b_a1 = s_ref[1]
    w_l1 = s_ref[2]
    b_l1 = s_ref[3]
    w_l2 = s_ref[4]
    b_l2 = s_ref[5]
    w_a2 = s_ref[6]
    b_a2 = s_ref[7]
    w_a3 = s_ref[8]
    b_a3 = s_ref[9]
    w_vh = s_ref[10]
    b_vh = s_ref[11]

    colsum = jnp.zeros((n,), jnp.float32)
    for k in range(_NCHUNK):
        _copy(k).wait()
        colsum = colsum + jnp.sum(dsm_vmem[pl.ds(k * ch, ch), :], axis=0)
    deg = colsum + 1.0
    dinv = jnp.where(deg > 0, jax.lax.rsqrt(deg), 0.0)

    y = w_a1 * x
    z = y * dinv[None, :]
    t = jnp.dot(z, dsm_vmem[...], preferred_element_type=jnp.float32)
    h = jnp.maximum(t * dinv[None, :] + y * (dinv * dinv)[None, :] + b_a1, 0.0)

    h = jnp.maximum(w_l1 * h + b_l1, 0.0)
    h = jnp.maximum(w_l2 * h + b_l2, 0.0)

    inv_np1 = 1.0 / (n + 1.0)
    y = w_a2 * h
    h = jnp.maximum((jnp.sum(y, axis=1, keepdims=True) + y) * inv_np1 + b_a2, 0.0)
    y = w_a3 * h
    h = jnp.maximum((jnp.sum(y, axis=1, keepdims=True) + y) * inv_np1 + b_a3, 0.0)

    ap_ref[...] = jnp.ones_like(x)

    m = jnp.max(h)
    sv_ref[...] = jnp.full((1, 1), w_vh * m + b_vh, jnp.float32)


def kernel(x, DSM, W_a1, b_a1, W_l1, b_l1, W_l2, b_l2, W_a2, b_a2,
           W_a3, b_a3, W_ah, b_ah, W_vh, b_vh):
    B, N = x.shape
    scal = jnp.stack([
        W_a1[0, 0], b_a1[0], W_l1[0, 0], b_l1[0], W_l2[0, 0], b_l2[0],
        W_a2[0, 0], b_a2[0], W_a3[0, 0], b_a3[0], W_vh[0, 0], b_vh[0],
    ]).astype(jnp.float32)

    ap, sv = pl.pallas_call(
        _policy_body,
        in_specs=[
            pl.BlockSpec(memory_space=pltpu.SMEM),
            pl.BlockSpec(memory_space=pltpu.VMEM),
            pl.BlockSpec(memory_space=pl.ANY),
        ],
        out_specs=[
            pl.BlockSpec(memory_space=pltpu.VMEM),
            pl.BlockSpec(memory_space=pltpu.VMEM),
        ],
        out_shape=[
            jax.ShapeDtypeStruct((B, N), jnp.float32),
            jax.ShapeDtypeStruct((1, 1), jnp.float32),
        ],
        scratch_shapes=[
            pltpu.VMEM((N, N), jnp.float32),
            pltpu.SemaphoreType.DMA((_NCHUNK,)),
        ],
    )(scal, x, DSM)

    return ap[:, :, None], sv[:, :, None]


# speedup vs baseline: 1.0161x; 1.0161x over previous
import jax
import jax.numpy as jnp
from jax.experimental import pallas as pl
from jax.experimental.pallas import tpu as pltpu

_NCHUNK = 8


def _policy_body(s_ref, x_ref, dsm_hbm, ap_ref, sv_ref, dsm_vmem, dsm_bf16, sems):
    n = dsm_vmem.shape[0]
    ch = n // _NCHUNK
    x = x_ref[...]

    def _copy(k):
        return pltpu.make_async_copy(
            dsm_hbm.at[pl.ds(k * ch, ch), :],
            dsm_vmem.at[pl.ds(k * ch, ch), :],
            sems.at[k],
        )

    for k in range(_NCHUNK):
        _copy(k).start()

    w_a1 = s_ref[0]
    b_a1 = s_ref[1]
    w_l1 = s_ref[2]
    b_l1 = s_ref[3]
    w_l2 = s_ref[4]
    b_l2 = s_ref[5]
    w_a2 = s_ref[6]
    b_a2 = s_ref[7]
    w_a3 = s_ref[8]
    b_a3 = s_ref[9]
    w_vh = s_ref[10]
    b_vh = s_ref[11]

    colsum = jnp.zeros((n,), jnp.float32)
    for k in range(_NCHUNK):
        _copy(k).wait()
        blk = dsm_vmem[pl.ds(k * ch, ch), :]
        colsum = colsum + jnp.sum(blk, axis=0)
        dsm_bf16[pl.ds(k * ch, ch), :] = blk.astype(jnp.bfloat16)
    deg = colsum + 1.0
    dinv = jnp.where(deg > 0, jax.lax.rsqrt(deg), 0.0)

    y = w_a1 * x
    z = y * dinv[None, :]
    t = jnp.dot(z.astype(jnp.bfloat16), dsm_bf16[...],
                preferred_element_type=jnp.float32)
    h = jnp.maximum(t * dinv[None, :] + y * (dinv * dinv)[None, :] + b_a1, 0.0)

    h = jnp.maximum(w_l1 * h + b_l1, 0.0)
    h = jnp.maximum(w_l2 * h + b_l2, 0.0)

    inv_np1 = 1.0 / (n + 1.0)
    y = w_a2 * h
    h = jnp.maximum((jnp.sum(y, axis=1, keepdims=True) + y) * inv_np1 + b_a2, 0.0)
    y = w_a3 * h
    h = jnp.maximum((jnp.sum(y, axis=1, keepdims=True) + y) * inv_np1 + b_a3, 0.0)

    ap_ref[...] = jnp.ones_like(x)

    m = jnp.max(h)
    sv_ref[...] = jnp.full((1, 1), w_vh * m + b_vh, jnp.float32)


def kernel(x, DSM, W_a1, b_a1, W_l1, b_l1, W_l2, b_l2, W_a2, b_a2,
           W_a3, b_a3, W_ah, b_ah, W_vh, b_vh):
    B, N = x.shape
    scal = jnp.stack([
        W_a1[0, 0], b_a1[0], W_l1[0, 0], b_l1[0], W_l2[0, 0], b_l2[0],
        W_a2[0, 0], b_a2[0], W_a3[0, 0], b_a3[0], W_vh[0, 0], b_vh[0],
    ]).astype(jnp.float32)

    ap, sv = pl.pallas_call(
        _policy_body,
        in_specs=[
            pl.BlockSpec(memory_space=pltpu.SMEM),
            pl.BlockSpec(memory_space=pltpu.VMEM),
            pl.BlockSpec(memory_space=pl.ANY),
        ],
        out_specs=[
            pl.BlockSpec(memory_space=pltpu.VMEM),
            pl.BlockSpec(memory_space=pltpu.VMEM),
        ],
        out_shape=[
            jax.ShapeDtypeStruct((B, N), jnp.float32),
            jax.ShapeDtypeStruct((1, 1), jnp.float32),
        ],
        scratch_shapes=[
            pltpu.VMEM((N, N), jnp.float32),
            pltpu.VMEM((N, N), jnp.bfloat16),
            pltpu.SemaphoreType.DMA((_NCHUNK,)),
        ],
    )(scal, x, DSM)

    return ap[:, :, None], sv[:, :, None]


# speedup vs baseline: 1.0576x; 1.0409x over previous
import jax
import jax.numpy as jnp
from jax.experimental import pallas as pl
from jax.experimental.pallas import tpu as pltpu

_NCHUNK = 8


def _policy_body(wa1_ref, ba1_ref, wl1_ref, bl1_ref, wl2_ref, bl2_ref,
                 wa2_ref, ba2_ref, wa3_ref, ba3_ref, wvh_ref, bvh_ref,
                 x_ref, dsm_hbm, ap_ref, sv_ref, dsm_vmem, dsm_bf16, sems):
    n = dsm_vmem.shape[0]
    ch = n // _NCHUNK
    x = x_ref[...]

    def _copy(k):
        return pltpu.make_async_copy(
            dsm_hbm.at[pl.ds(k * ch, ch), :],
            dsm_vmem.at[pl.ds(k * ch, ch), :],
            sems.at[k],
        )

    for k in range(_NCHUNK):
        _copy(k).start()

    w_a1 = wa1_ref[0, 0]
    b_a1 = ba1_ref[0]
    w_l1 = wl1_ref[0, 0]
    b_l1 = bl1_ref[0]
    w_l2 = wl2_ref[0, 0]
    b_l2 = bl2_ref[0]
    w_a2 = wa2_ref[0, 0]
    b_a2 = ba2_ref[0]
    w_a3 = wa3_ref[0, 0]
    b_a3 = ba3_ref[0]
    w_vh = wvh_ref[0, 0]
    b_vh = bvh_ref[0]

    colsum = jnp.zeros((n,), jnp.float32)
    for k in range(_NCHUNK):
        _copy(k).wait()
        blk = dsm_vmem[pl.ds(k * ch, ch), :]
        colsum = colsum + jnp.sum(blk, axis=0)
        dsm_bf16[pl.ds(k * ch, ch), :] = blk.astype(jnp.bfloat16)
    deg = colsum + 1.0
    dinv = jnp.where(deg > 0, jax.lax.rsqrt(deg), 0.0)

    y = w_a1 * x
    z = y * dinv[None, :]
    t = jnp.dot(z.astype(jnp.bfloat16), dsm_bf16[...],
                preferred_element_type=jnp.float32)
    h = jnp.maximum(t * dinv[None, :] + y * (dinv * dinv)[None, :] + b_a1, 0.0)

    h = jnp.maximum(w_l1 * h + b_l1, 0.0)
    h = jnp.maximum(w_l2 * h + b_l2, 0.0)

    inv_np1 = 1.0 / (n + 1.0)
    y = w_a2 * h
    h = jnp.maximum((jnp.sum(y, axis=1, keepdims=True) + y) * inv_np1 + b_a2, 0.0)
    y = w_a3 * h
    h = jnp.maximum((jnp.sum(y, axis=1, keepdims=True) + y) * inv_np1 + b_a3, 0.0)

    ap_ref[...] = jnp.ones_like(x)

    m = jnp.max(h)
    sv_ref[...] = jnp.full((1, 1), w_vh * m + b_vh, jnp.float32)


def kernel(x, DSM, W_a1, b_a1, W_l1, b_l1, W_l2, b_l2, W_a2, b_a2,
           W_a3, b_a3, W_ah, b_ah, W_vh, b_vh):
    B, N = x.shape
    smem = pl.BlockSpec(memory_space=pltpu.SMEM)

    ap, sv = pl.pallas_call(
        _policy_body,
        in_specs=[smem] * 12 + [
            pl.BlockSpec(memory_space=pltpu.VMEM),
            pl.BlockSpec(memory_space=pl.ANY),
        ],
        out_specs=[
            pl.BlockSpec(memory_space=pltpu.VMEM),
            pl.BlockSpec(memory_space=pltpu.VMEM),
        ],
        out_shape=[
            jax.ShapeDtypeStruct((B, N), jnp.float32),
            jax.ShapeDtypeStruct((1, 1), jnp.float32),
        ],
        scratch_shapes=[
            pltpu.VMEM((N, N), jnp.float32),
            pltpu.VMEM((N, N), jnp.bfloat16),
            pltpu.SemaphoreType.DMA((_NCHUNK,)),
        ],
    )(W_a1, b_a1, W_l1, b_l1, W_l2, b_l2, W_a2, b_a2, W_a3, b_a3,
      W_vh, b_vh, x, DSM)

    return ap[:, :, None], sv[:, :, None]
